# baseline (speedup 1.0000x reference)
import jax
import jax.numpy as jnp
from jax import lax
from jax.experimental import pallas as pl
from jax.experimental.pallas import tpu as pltpu
from jax.experimental.pallas import tpu_sc as plsc

D = 32
B = 16384
H = 50
NC, NS, L = 2, 16, 16
NW = NC * NS
EPW = B // NW
CHUNK = 128
STEPS = EPW // CHUNK
RPC = CHUNK * H
VL = CHUNK // L


def _body(pred_hbm, table_hbm, out_hbm, raw_v, idxs_v, acc_v, psem, gsem0,
          gsem1, osem0, osem1):
    wid = lax.axis_index("s") * NC + lax.axis_index("c")
    ebase = wid * EPW
    gsems = [gsem0, gsem1]
    osems = [osem0, osem1]
    scale = jnp.float32(1.0 / H)
    zv = jnp.zeros((L,), jnp.float32)
    lanes_h = jnp.arange(L, dtype=jnp.int32) * H

    def stage(s):
        b = s & 1
        e0 = ebase + s * CHUNK
        return pltpu.async_copy(
            pred_hbm.at[pl.ds(e0 * H, RPC)], raw_v.at[b], psem
        )

    def transpose_clamp(s):
        b = s & 1

        def row(j, c):
            for k in range(VL):
                iv = lanes_h + (k * L * H + j)
                v = plsc.load_gather(raw_v.at[b], [iv])
                idxs_v[b, j, pl.ds(k * L, L)] = jnp.maximum(v, 0)
            return c

        lax.fori_loop(0, H, row, 0)

    def zero(s):
        b = s & 1

        def row(e, c):
            acc_v[b, e, pl.ds(0, L)] = zv
            acc_v[b, e, pl.ds(L, L)] = zv
            return c

        lax.fori_loop(0, CHUNK, row, 0)

    def fire(s):
        b = s & 1

        def one(j, c):
            pltpu.async_copy(
                table_hbm.at[idxs_v.at[b, j]], acc_v.at[b], gsems[b],
                add=True,
            )
            return c

        lax.fori_loop(0, H, one, 0)

    def drain(s):
        b = s & 1

        def one(j, c):
            pltpu.make_async_copy(
                table_hbm.at[idxs_v.at[b, j]], acc_v.at[b], gsems[b]
            ).wait()
            return c

        lax.fori_loop(0, H, one, 0)

    def scale_rows(s):
        b = s & 1

        def row(e, c):
            acc_v[b, e, pl.ds(0, L)] = acc_v[b, e, pl.ds(0, L)] * scale
            acc_v[b, e, pl.ds(L, L)] = acc_v[b, e, pl.ds(L, L)] * scale
            return c

        lax.fori_loop(0, CHUNK, row, 0)

    def write(s):
        b = s & 1
        e0 = ebase + s * CHUNK
        return pltpu.async_copy(acc_v.at[b], out_hbm.at[pl.ds(e0, CHUNK)],
                                osems[b])

    pcp = [None] * STEPS
    ocp = [None] * STEPS
    pcp[0] = stage(0)
    for s in range(STEPS):
        pcp[s].wait()
        transpose_clamp(s)
        if s + 1 < STEPS:
            pcp[s + 1] = stage(s + 1)
        if s >= 2:
            ocp[s - 2].wait()
        zero(s)
        fire(s)
        if s >= 1:
            drain(s - 1)
            scale_rows(s - 1)
            ocp[s - 1] = write(s - 1)
    drain(STEPS - 1)
    scale_rows(STEPS - 1)
    ocp[STEPS - 1] = write(STEPS - 1)
    ocp[STEPS - 2].wait()
    ocp[STEPS - 1].wait()


def kernel(idx, table):
    pred = idx[:, :, 0].reshape(B * H)
    k = pl.kernel(
        _body,
        out_type=jax.ShapeDtypeStruct((B, D), jnp.float32),
        mesh=plsc.VectorSubcoreMesh(core_axis_name="c", subcore_axis_name="s"),
        scratch_types=[
            pltpu.VMEM((2, RPC), jnp.int32),
            pltpu.VMEM((2, H, CHUNK), jnp.int32),
            pltpu.VMEM((2, CHUNK, D), jnp.float32),
            pltpu.SemaphoreType.DMA,
            pltpu.SemaphoreType.DMA,
            pltpu.SemaphoreType.DMA,
            pltpu.SemaphoreType.DMA,
            pltpu.SemaphoreType.DMA,
        ],
        compiler_params=pltpu.CompilerParams(
            needs_layout_passes=False, use_tc_tiling_on_sc=False
        ),
    )
    return k(pred, table)


# speedup vs baseline: 1.0006x; 1.0006x over previous
import jax
import jax.numpy as jnp
from jax import lax
from jax.experimental import pallas as pl
from jax.experimental.pallas import tpu as pltpu
from jax.experimental.pallas import tpu_sc as plsc

D = 32
B = 16384
H = 50
NC, NS, L = 2, 16, 16
NW = NC * NS
EPW = B // NW
CHUNK = 128
STEPS = EPW // CHUNK
RPC = CHUNK * H
VL = CHUNK // L


def _body(pred_hbm, table_hbm, out_hbm, raw_v, idxs_v, acc_v, psem, gsem0,
          gsem1, osem0, osem1):
    wid = lax.axis_index("s") * NC + lax.axis_index("c")
    ebase = wid * EPW
    gsems = [gsem0, gsem1]
    osems = [osem0, osem1]
    scale = jnp.float32(1.0 / H)
    zv = jnp.zeros((L,), jnp.float32)
    lanes_h = jnp.arange(L, dtype=jnp.int32) * H

    def stage(s):
        b = s & 1
        e0 = ebase + s * CHUNK
        return pltpu.async_copy(
            pred_hbm.at[pl.ds(e0 * H, RPC)], raw_v.at[b], psem
        )

    def transpose_clamp(s):
        b = s & 1

        def row(j, c):
            for k in range(VL):
                iv = lanes_h + (k * L * H + j)
                v = plsc.load_gather(raw_v.at[b], [iv])
                idxs_v[b, j, pl.ds(k * L, L)] = jnp.maximum(v, 0)
            return c

        lax.fori_loop(0, H, row, 0)

    def zero(s):
        b = s & 1

        def row(e, c):
            acc_v[b, e, pl.ds(0, L)] = zv
            acc_v[b, e, pl.ds(L, L)] = zv
            return c

        lax.fori_loop(0, CHUNK, row, 0)

    def fire(s):
        b = s & 1

        def one(j, c):
            pltpu.async_copy(
                table_hbm.at[idxs_v.at[b, j]], acc_v.at[b], gsems[b],
                add=True,
            )
            return c

        lax.fori_loop(0, H, one, 0)

    def drain(s):
        b = s & 1

        def one(j, c):
            pltpu.make_async_copy(
                table_hbm.at[idxs_v.at[b, j]], acc_v.at[b], gsems[b]
            ).wait()
            return c

        lax.fori_loop(0, H, one, 0)

    def scale_rows(s):
        b = s & 1

        def row(e, c):
            acc_v[b, e, pl.ds(0, L)] = acc_v[b, e, pl.ds(0, L)] * scale
            acc_v[b, e, pl.ds(L, L)] = acc_v[b, e, pl.ds(L, L)] * scale
            return c

        lax.fori_loop(0, CHUNK, row, 0)

    def write(s):
        b = s & 1
        e0 = ebase + s * CHUNK
        return pltpu.async_copy(acc_v.at[b], out_hbm.at[pl.ds(e0, CHUNK)],
                                osems[b])

    pcp = [None] * STEPS
    ocp = [None] * STEPS
    pcp[0] = stage(0)
    for s in range(STEPS):
        pcp[s].wait()
        transpose_clamp(s)
        if s + 1 < STEPS:
            pcp[s + 1] = stage(s + 1)
        if s >= 2:
            ocp[s - 2].wait()
        zero(s)
        fire(s)
        if s >= 1:
            drain(s - 1)
            scale_rows(s - 1)
            ocp[s - 1] = write(s - 1)
    drain(STEPS - 1)
    scale_rows(STEPS - 1)
    ocp[STEPS - 1] = write(STEPS - 1)
    ocp[STEPS - 2].wait()
    ocp[STEPS - 1].wait()


def kernel(idx, table):
    pred = jnp.maximum(idx[:, :, 0], 0).reshape(B * H)
    k = pl.kernel(
        _body,
        out_type=jax.ShapeDtypeStruct((B, D), jnp.float32),
        mesh=plsc.VectorSubcoreMesh(core_axis_name="c", subcore_axis_name="s"),
        scratch_types=[
            pltpu.VMEM((2, RPC), jnp.int32),
            pltpu.VMEM((2, H, CHUNK), jnp.int32),
            pltpu.VMEM((2, CHUNK, D), jnp.float32),
            pltpu.SemaphoreType.DMA,
            pltpu.SemaphoreType.DMA,
            pltpu.SemaphoreType.DMA,
            pltpu.SemaphoreType.DMA,
            pltpu.SemaphoreType.DMA,
        ],
        compiler_params=pltpu.CompilerParams(
            needs_layout_passes=False, use_tc_tiling_on_sc=False
        ),
    )
    return k(pred, table)


# speedup vs baseline: 1.0007x; 1.0001x over previous
import jax
import jax.numpy as jnp
from jax import lax
from jax.experimental import pallas as pl
from jax.experimental.pallas import tpu as pltpu
from jax.experimental.pallas import tpu_sc as plsc

D = 32
B = 16384
H = 50
NC, NS, L = 2, 16, 16
NW = NC * NS
EPW = B // NW
CHUNK = 128
STEPS = EPW // CHUNK
RPC = CHUNK * H
VL = CHUNK // L


def _body(pred_hbm, table_hbm, out_hbm, raw_v, idxs_v, acc_v, psem, gsem0,
          gsem1, osem0, osem1):
    wid = lax.axis_index("s") * NC + lax.axis_index("c")
    ebase = wid * EPW
    gsems = [gsem0, gsem1]
    osems = [osem0, osem1]
    scale = jnp.float32(1.0 / H)
    zv = jnp.zeros((L,), jnp.float32)
    lanes_h = jnp.arange(L, dtype=jnp.int32) * H

    def stage(s):
        b = s & 1
        e0 = ebase + s * CHUNK
        return pltpu.async_copy(
            pred_hbm.at[pl.ds(e0 * H, RPC)], raw_v.at[b], psem
        )

    def transpose_clamp(s):
        b = s & 1

        def row(j, c):
            for k in range(VL):
                iv = lanes_h + (k * L * H + j)
                v = plsc.load_gather(raw_v.at[b], [iv])
                idxs_v[b, j, pl.ds(k * L, L)] = jnp.maximum(v, 0)
            return c

        lax.fori_loop(0, H, row, 0)

    def zero(s):
        b = s & 1

        def row(e, c):
            acc_v[b, e, pl.ds(0, L)] = zv
            acc_v[b, e, pl.ds(L, L)] = zv
            return c

        lax.fori_loop(0, CHUNK, row, 0)

    def fire(s):
        b = s & 1

        def one(j, c):
            pltpu.async_copy(
                table_hbm.at[idxs_v.at[b, j]], acc_v.at[b], gsems[b],
                add=True,
            )
            return c

        lax.fori_loop(0, H, one, 0)

    def drain(s):
        b = s & 1

        def one(j, c):
            pltpu.make_async_copy(
                table_hbm.at[idxs_v.at[b, j]], acc_v.at[b], gsems[b]
            ).wait()
            return c

        lax.fori_loop(0, H, one, 0)

    def scale_rows(s):
        b = s & 1

        def row(e, c):
            acc_v[b, e, pl.ds(0, L)] = acc_v[b, e, pl.ds(0, L)] * scale
            acc_v[b, e, pl.ds(L, L)] = acc_v[b, e, pl.ds(L, L)] * scale
            return c

        lax.fori_loop(0, CHUNK, row, 0)

    def write(s):
        b = s & 1
        e0 = ebase + s * CHUNK
        return pltpu.async_copy(acc_v.at[b], out_hbm.at[pl.ds(e0, CHUNK)],
                                osems[b])

    pcp = [None] * STEPS
    ocp = [None] * STEPS
    pcp[0] = stage(0)
    for s in range(STEPS):
        pcp[s].wait()
        transpose_clamp(s)
        if s + 1 < STEPS:
            pcp[s + 1] = stage(s + 1)
        if s >= 2:
            ocp[s - 2].wait()
        zero(s)
        fire(s)
        if s >= 1:
            drain(s - 1)
            scale_rows(s - 1)
            ocp[s - 1] = write(s - 1)
    drain(STEPS - 1)
    scale_rows(STEPS - 1)
    ocp[STEPS - 1] = write(STEPS - 1)
    ocp[STEPS - 2].wait()
    ocp[STEPS - 1].wait()


def kernel(idx, table):
    pred = jnp.maximum(idx[:, :, 0], 0).reshape(B * H)
    table = lax.optimization_barrier(table.reshape(250000, 128))
    table = table.reshape(1000000, D)
    k = pl.kernel(
        _body,
        out_type=jax.ShapeDtypeStruct((B, D), jnp.float32),
        mesh=plsc.VectorSubcoreMesh(core_axis_name="c", subcore_axis_name="s"),
        scratch_types=[
            pltpu.VMEM((2, RPC), jnp.int32),
            pltpu.VMEM((2, H, CHUNK), jnp.int32),
            pltpu.VMEM((2, CHUNK, D), jnp.float32),
            pltpu.SemaphoreType.DMA,
            pltpu.SemaphoreType.DMA,
            pltpu.SemaphoreType.DMA,
            pltpu.SemaphoreType.DMA,
            pltpu.SemaphoreType.DMA,
        ],
        compiler_params=pltpu.CompilerParams(
            needs_layout_passes=False, use_tc_tiling_on_sc=False
        ),
    )
    return k(pred, table)


# speedup vs baseline: 1.0278x; 1.0271x over previous
import jax
import jax.numpy as jnp
from jax import lax
from jax.experimental import pallas as pl
from jax.experimental.pallas import tpu as pltpu
from jax.experimental.pallas import tpu_sc as plsc

D = 32
B = 16384
H = 50
NC, NS, L = 2, 16, 16
NW = NC * NS
EPW = B // NW
CHUNK = 128
STEPS = EPW // CHUNK
RPC = CHUNK * H
VL = CHUNK // L


def _body(pred_hbm, table_hbm, out_hbm, raw_v, idxs_v, acc_v, psem, gsem0,
          gsem1, osem0, osem1):
    wid = lax.axis_index("s") * NC + lax.axis_index("c")
    ebase = wid * EPW
    gsems = [gsem0, gsem1]
    osems = [osem0, osem1]
    scale = jnp.float32(1.0 / H)
    zv = jnp.zeros((L,), jnp.float32)
    lanes_h = jnp.arange(L, dtype=jnp.int32) * H

    def stage(s):
        b = s & 1
        e0 = ebase + s * CHUNK
        return pltpu.async_copy(
            pred_hbm.at[pl.ds(e0 * H, RPC)], raw_v.at[b], psem
        )

    def transpose_clamp(s):
        b = s & 1

        def row(j, c):
            for k in range(VL):
                iv = lanes_h + (k * L * H + j)
                v = plsc.load_gather(raw_v.at[b], [iv])
                v = jnp.maximum(v, 0)
                idxs_v[b, j, pl.ds(k * L, L)] = ((v >> 3) << 5) | ((v & 7) << 2)
            return c

        lax.fori_loop(0, H, row, 0)

    def zero(s):
        b = s & 1

        def row(e, c):
            acc_v[b, e, pl.ds(0, L)] = zv
            acc_v[b, e, pl.ds(L, L)] = zv
            return c

        lax.fori_loop(0, CHUNK, row, 0)

    def fire(s):
        b = s & 1

        def one(j, c):
            pltpu.async_copy(
                table_hbm.at[idxs_v.at[b, j]], acc_v.at[b], gsems[b],
                add=True,
            )
            return c

        lax.fori_loop(0, H, one, 0)

    def drain(s):
        b = s & 1

        def one(j, c):
            pltpu.make_async_copy(
                table_hbm.at[idxs_v.at[b, j]], acc_v.at[b], gsems[b]
            ).wait()
            return c

        lax.fori_loop(0, H, one, 0)

    def scale_rows(s):
        b = s & 1

        def row(e, c):
            acc_v[b, e, pl.ds(0, L)] = acc_v[b, e, pl.ds(0, L)] * scale
            acc_v[b, e, pl.ds(L, L)] = acc_v[b, e, pl.ds(L, L)] * scale
            return c

        lax.fori_loop(0, CHUNK, row, 0)

    def write(s):
        b = s & 1
        e0 = ebase + s * CHUNK
        return pltpu.async_copy(acc_v.at[b], out_hbm.at[pl.ds(e0, CHUNK)],
                                osems[b])

    pcp = [None] * STEPS
    ocp = [None] * STEPS
    pcp[0] = stage(0)
    for s in range(STEPS):
        pcp[s].wait()
        transpose_clamp(s)
        if s + 1 < STEPS:
            pcp[s + 1] = stage(s + 1)
        if s >= 2:
            ocp[s - 2].wait()
        zero(s)
        fire(s)
        if s >= 1:
            drain(s - 1)
            scale_rows(s - 1)
            ocp[s - 1] = write(s - 1)
    drain(STEPS - 1)
    scale_rows(STEPS - 1)
    ocp[STEPS - 1] = write(STEPS - 1)
    ocp[STEPS - 2].wait()
    ocp[STEPS - 1].wait()


def kernel(idx, table):
    pred = jnp.maximum(idx[:, :, 0], 0).reshape(B * H)
    table = jnp.pad(table, ((0, 0), (0, 128 - D))).reshape(4 * 1000000, D)
    k = pl.kernel(
        _body,
        out_type=jax.ShapeDtypeStruct((B, D), jnp.float32),
        mesh=plsc.VectorSubcoreMesh(core_axis_name="c", subcore_axis_name="s"),
        scratch_types=[
            pltpu.VMEM((2, RPC), jnp.int32),
            pltpu.VMEM((2, H, CHUNK), jnp.int32),
            pltpu.VMEM((2, CHUNK, D), jnp.float32),
            pltpu.SemaphoreType.DMA,
            pltpu.SemaphoreType.DMA,
            pltpu.SemaphoreType.DMA,
            pltpu.SemaphoreType.DMA,
            pltpu.SemaphoreType.DMA,
        ],
        compiler_params=pltpu.CompilerParams(
            needs_layout_passes=False, use_tc_tiling_on_sc=False
        ),
    )
    return k(pred, table)
